# Initial kernel scaffold; baseline (speedup 1.0000x reference)
import jax
import jax.numpy as jnp
from jax.experimental import pallas as pl


def kernel(token_ids, weight):
    raise NotImplementedError("write your pallas kernel here")


# speedup vs baseline: 2.9740x; 2.9740x over previous
import functools

import jax
import jax.numpy as jnp
from jax import lax
from jax.experimental import pallas as pl
from jax.experimental.pallas import tpu as pltpu
from jax.experimental.pallas import tpu_sc as plsc

NUM_EMBEDDINGS = 100000
EMBEDDING_DIM = 128

B, L = 4096, 50
TOTAL = B * L
NC, NS = 2, 16
NW = NC * NS
PER_W = TOTAL // NW
CHUNK = 128
NCHUNK = PER_W // CHUNK


def _sc_gather(idx_flat, weight):
    mesh = plsc.VectorSubcoreMesh(
        core_axis_name="c", subcore_axis_name="s", num_cores=NC, num_subcores=NS
    )

    @functools.partial(
        pl.kernel,
        out_type=jax.ShapeDtypeStruct((TOTAL, EMBEDDING_DIM), jnp.float32),
        mesh=mesh,
        scratch_types=[
            pltpu.VMEM((NCHUNK, CHUNK), jnp.int32),
            pltpu.VMEM((CHUNK, EMBEDDING_DIM), jnp.float32),
            pltpu.SemaphoreType.DMA,
        ],
    )
    def k(idx_hbm, table_hbm, out_hbm, idx_v, rows_v, sem):
        wid = lax.axis_index("s") * NC + lax.axis_index("c")
        base = wid * PER_W
        pltpu.sync_copy(idx_hbm.at[wid], idx_v)

        @pl.loop(0, NCHUNK)
        def _(j):
            pltpu.async_copy(table_hbm.at[idx_v.at[j]], rows_v, sem).wait()
            pltpu.sync_copy(rows_v, out_hbm.at[pl.ds(base + j * CHUNK, CHUNK)])

    return k(idx_flat, weight)


def kernel(token_ids, weight):
    idx = token_ids.reshape(NW, NCHUNK, CHUNK).astype(jnp.int32)
    out = _sc_gather(idx, weight)
    return out.reshape(B, L, EMBEDDING_DIM)


# speedup vs baseline: 3.3616x; 1.1303x over previous
import functools

import jax
import jax.numpy as jnp
from jax import lax
from jax.experimental import pallas as pl
from jax.experimental.pallas import tpu as pltpu
from jax.experimental.pallas import tpu_sc as plsc

NUM_EMBEDDINGS = 100000
EMBEDDING_DIM = 128

B, L = 4096, 50
TOTAL = B * L
NC, NS = 2, 16
NW = NC * NS
PER_W = TOTAL // NW
CHUNK = 128
NCHUNK = PER_W // CHUNK
NBUF = 5
PF = 3


def _sc_gather(idx_flat, weight):
    mesh = plsc.VectorSubcoreMesh(
        core_axis_name="c", subcore_axis_name="s", num_cores=NC, num_subcores=NS
    )

    @functools.partial(
        pl.kernel,
        out_type=jax.ShapeDtypeStruct((TOTAL, EMBEDDING_DIM), jnp.float32),
        mesh=mesh,
        scratch_types=[
            pltpu.VMEM((NCHUNK, CHUNK), jnp.int32),
            pltpu.VMEM((NBUF, CHUNK, EMBEDDING_DIM), jnp.float32),
            pltpu.SemaphoreType.DMA,
            pltpu.SemaphoreType.DMA,
        ],
    )
    def k(idx_hbm, table_hbm, out_hbm, idx_v, rows_v, sem_in, sem_out):
        wid = lax.axis_index("s") * NC + lax.axis_index("c")
        base = wid * PER_W
        pltpu.sync_copy(idx_hbm.at[wid], idx_v)

        def start_gather(j, b):
            pltpu.async_copy(table_hbm.at[idx_v.at[j]], rows_v.at[b], sem_in)

        def wait_gather(b):
            pltpu.make_async_copy(
                table_hbm.at[idx_v.at[0]], rows_v.at[b], sem_in
            ).wait()

        def start_out(j, b):
            pltpu.async_copy(
                rows_v.at[b], out_hbm.at[pl.ds(base + j * CHUNK, CHUNK)], sem_out
            )

        def wait_out(b):
            pltpu.make_async_copy(
                rows_v.at[b], out_hbm.at[pl.ds(base, CHUNK)], sem_out
            ).wait()

        for j in range(PF):
            start_gather(j, j % NBUF)

        @pl.loop(0, NCHUNK, step=NBUF)
        def _(g):
            for b in range(NBUF):
                j = g + b
                wait_gather(b)
                start_out(j, b)

                @pl.when(jnp.logical_and(j + PF < NCHUNK, j >= NBUF - PF))
                def _():
                    wait_out((b + PF) % NBUF)

                @pl.when(j + PF < NCHUNK)
                def _():
                    start_gather(j + PF, (b + PF) % NBUF)

        for b in range(NBUF):
            wait_out(b)

    return k(idx_flat, weight)


def kernel(token_ids, weight):
    idx = token_ids.reshape(NW, NCHUNK, CHUNK).astype(jnp.int32)
    out = _sc_gather(idx, weight)
    return out.reshape(B, L, EMBEDDING_DIM)


# speedup vs baseline: 6.0576x; 1.8020x over previous
import functools

import jax
import jax.numpy as jnp
from jax import lax
from jax.experimental import pallas as pl
from jax.experimental.pallas import tpu as pltpu
from jax.experimental.pallas import tpu_sc as plsc

NUM_EMBEDDINGS = 100000
EMBEDDING_DIM = 128

B, L = 4096, 50
TOTAL = B * L
NC, NS = 2, 16
NW = NC * NS
SLAB_PER_W = B // NW
CHUNK = 2 * L
CHUNK_PAD = 104
NCHUNK = SLAB_PER_W // 2
IDX_PER_W = NCHUNK * CHUNK_PAD
NBUF = 8
PF = 5


def _sc_gather(idx_flat, weight):
    mesh = plsc.VectorSubcoreMesh(
        core_axis_name="c", subcore_axis_name="s", num_cores=NC, num_subcores=NS
    )

    @functools.partial(
        pl.kernel,
        out_type=jax.ShapeDtypeStruct((B, L, EMBEDDING_DIM), jnp.float32),
        mesh=mesh,
        scratch_types=[
            pltpu.VMEM((IDX_PER_W,), jnp.int32),
            pltpu.VMEM((NBUF, CHUNK, EMBEDDING_DIM), jnp.float32),
            pltpu.SemaphoreType.DMA,
            pltpu.SemaphoreType.DMA,
        ],
    )
    def k(idx_hbm, table_hbm, out_hbm, idx_v, rows_v, sem_in, sem_out):
        wid = lax.axis_index("s") * NC + lax.axis_index("c")
        slab0 = wid * SLAB_PER_W
        pltpu.sync_copy(idx_hbm.at[pl.ds(wid * IDX_PER_W, IDX_PER_W)], idx_v)

        def start_gather(j, b):
            pltpu.async_copy(
                table_hbm.at[idx_v.at[pl.ds(j * CHUNK_PAD, CHUNK)]],
                rows_v.at[b],
                sem_in,
            )

        def wait_gather(b):
            pltpu.make_async_copy(
                table_hbm.at[idx_v.at[pl.ds(0, CHUNK)]], rows_v.at[b], sem_in
            ).wait()

        def start_out(j, b):
            pltpu.async_copy(
                rows_v.at[b, pl.ds(0, L)], out_hbm.at[slab0 + 2 * j], sem_out
            )
            pltpu.async_copy(
                rows_v.at[b, pl.ds(L, L)], out_hbm.at[slab0 + 2 * j + 1], sem_out
            )

        def wait_out(b):
            for h in range(2):
                pltpu.make_async_copy(
                    rows_v.at[b, pl.ds(h * L, L)], out_hbm.at[slab0], sem_out
                ).wait()

        for j in range(PF):
            start_gather(j, j % NBUF)

        @pl.loop(0, NCHUNK, step=NBUF)
        def _(g):
            for b in range(NBUF):
                j = g + b
                wait_gather(b)
                start_out(j, b)

                @pl.when(jnp.logical_and(j + PF < NCHUNK, j >= NBUF - PF))
                def _():
                    wait_out((b + PF) % NBUF)

                @pl.when(j + PF < NCHUNK)
                def _():
                    start_gather(j + PF, (b + PF) % NBUF)

        for b in range(NBUF):
            wait_out(b)

    return k(idx_flat, weight)


def kernel(token_ids, weight):
    idx = token_ids.reshape(B // 2, CHUNK).astype(jnp.int32)
    idx = jnp.pad(idx, ((0, 0), (0, CHUNK_PAD - CHUNK)))
    return _sc_gather(idx.reshape(-1), weight)


# speedup vs baseline: 10.7038x; 1.7670x over previous
import functools

import jax
import jax.numpy as jnp
from jax import lax
from jax.experimental import pallas as pl
from jax.experimental.pallas import tpu as pltpu
from jax.experimental.pallas import tpu_sc as plsc

NUM_EMBEDDINGS = 100000
EMBEDDING_DIM = 128

B, L = 4096, 50
TOTAL = B * L
NC, NS = 2, 16
NW = NC * NS
PER_W = TOTAL // NW
CHUNK = 128
NCHUNK = PER_W // CHUNK
NBUF = 5
PF = 3


def _sc_gather(idx_flat, weight):
    mesh = plsc.VectorSubcoreMesh(
        core_axis_name="c", subcore_axis_name="s", num_cores=NC, num_subcores=NS
    )

    @functools.partial(
        pl.kernel,
        out_type=jax.ShapeDtypeStruct((TOTAL, EMBEDDING_DIM), jnp.float32),
        mesh=mesh,
        scratch_types=[
            pltpu.VMEM((PER_W,), jnp.int32),
            pltpu.VMEM((NBUF, CHUNK, EMBEDDING_DIM), jnp.float32),
            pltpu.SemaphoreType.DMA,
            pltpu.SemaphoreType.DMA,
        ],
    )
    def k(idx_hbm, table_hbm, out_hbm, idx_v, rows_v, sem_in, sem_out):
        wid = lax.axis_index("s") * NC + lax.axis_index("c")
        base = wid * PER_W
        pltpu.sync_copy(idx_hbm.at[pl.ds(base, PER_W)], idx_v)

        def start_gather(j, b):
            pltpu.async_copy(
                table_hbm.at[idx_v.at[pl.ds(j * CHUNK, CHUNK)]], rows_v.at[b], sem_in
            )

        def wait_gather(b):
            pltpu.make_async_copy(
                table_hbm.at[idx_v.at[pl.ds(0, CHUNK)]], rows_v.at[b], sem_in
            ).wait()

        def start_out(j, b):
            pltpu.async_copy(
                rows_v.at[b], out_hbm.at[pl.ds(base + j * CHUNK, CHUNK)], sem_out
            )

        def wait_out(b):
            pltpu.make_async_copy(
                rows_v.at[b], out_hbm.at[pl.ds(base, CHUNK)], sem_out
            ).wait()

        for j in range(PF):
            start_gather(j, j % NBUF)

        @pl.loop(0, NCHUNK, step=NBUF)
        def _(g):
            for b in range(NBUF):
                j = g + b
                wait_gather(b)
                start_out(j, b)

                @pl.when(jnp.logical_and(j + PF < NCHUNK, j >= NBUF - PF))
                def _():
                    wait_out((b + PF) % NBUF)

                @pl.when(j + PF < NCHUNK)
                def _():
                    start_gather(j + PF, (b + PF) % NBUF)

        for b in range(NBUF):
            wait_out(b)

    return k(idx_flat, weight)


def kernel(token_ids, weight):
    idx = token_ids.astype(jnp.int32).T.reshape(TOTAL)
    out = _sc_gather(idx, weight)
    return out.reshape(L, B, EMBEDDING_DIM).transpose(1, 0, 2)


# speedup vs baseline: 10.7414x; 1.0035x over previous
import functools

import jax
import jax.numpy as jnp
from jax import lax
from jax.experimental import pallas as pl
from jax.experimental.pallas import tpu as pltpu
from jax.experimental.pallas import tpu_sc as plsc

NUM_EMBEDDINGS = 100000
EMBEDDING_DIM = 128

B, L = 4096, 50
TOTAL = B * L
NC, NS = 2, 16
NW = NC * NS
PER_W = TOTAL // NW
CHUNK = 128
NCHUNK = PER_W // CHUNK
NBUF = 5
PF = 4


def _sc_gather(idx_flat, weight):
    mesh = plsc.VectorSubcoreMesh(
        core_axis_name="c", subcore_axis_name="s", num_cores=NC, num_subcores=NS
    )

    @functools.partial(
        pl.kernel,
        out_type=jax.ShapeDtypeStruct((TOTAL, EMBEDDING_DIM), jnp.float32),
        mesh=mesh,
        scratch_types=[
            pltpu.VMEM((PER_W,), jnp.int32),
            pltpu.VMEM((NBUF, CHUNK, EMBEDDING_DIM), jnp.float32),
            pltpu.SemaphoreType.DMA,
            pltpu.SemaphoreType.DMA,
        ],
    )
    def k(idx_hbm, table_hbm, out_hbm, idx_v, rows_v, sem_in, sem_out):
        wid = lax.axis_index("s") * NC + lax.axis_index("c")
        base = wid * PER_W
        pltpu.sync_copy(idx_hbm.at[pl.ds(base, PER_W)], idx_v)

        def start_gather(j, b):
            pltpu.async_copy(
                table_hbm.at[idx_v.at[pl.ds(j * CHUNK, CHUNK)]], rows_v.at[b], sem_in
            )

        def wait_gather(b):
            pltpu.make_async_copy(
                table_hbm.at[idx_v.at[pl.ds(0, CHUNK)]], rows_v.at[b], sem_in
            ).wait()

        def start_out(j, b):
            pltpu.async_copy(
                rows_v.at[b], out_hbm.at[pl.ds(base + j * CHUNK, CHUNK)], sem_out
            )

        def wait_out(b):
            pltpu.make_async_copy(
                rows_v.at[b], out_hbm.at[pl.ds(base, CHUNK)], sem_out
            ).wait()

        for j in range(PF):
            start_gather(j, j % NBUF)

        @pl.loop(0, NCHUNK, step=NBUF)
        def _(g):
            for b in range(NBUF):
                j = g + b
                wait_gather(b)
                start_out(j, b)

                @pl.when(jnp.logical_and(j + PF < NCHUNK, j >= NBUF - PF))
                def _():
                    wait_out((b + PF) % NBUF)

                @pl.when(j + PF < NCHUNK)
                def _():
                    start_gather(j + PF, (b + PF) % NBUF)

        for b in range(NBUF):
            wait_out(b)

    return k(idx_flat, weight)


def kernel(token_ids, weight):
    idx = token_ids.astype(jnp.int32).T.reshape(TOTAL)
    out = _sc_gather(idx, weight)
    return out.reshape(L, B, EMBEDDING_DIM).transpose(1, 0, 2)
